# Initial kernel scaffold; baseline (speedup 1.0000x reference)
import jax
import jax.numpy as jnp
from jax.experimental import pallas as pl


def kernel(frame_embeds, W, b):
    raise NotImplementedError("write your pallas kernel here")


# speedup vs baseline: 13.4929x; 13.4929x over previous
import numpy as np
import jax
import jax.numpy as jnp
from jax.experimental import pallas as pl

_BS, _T, _Q, _C = 8, 32, 8, 256
_N = _BS * _Q
_TOPK = 5


def _build_consts():
    try:
        cpu = jax.devices("cpu")[0]
        with jax.default_device(cpu):
            br = np.asarray(
                jax.random.randint(jax.random.key(42), (_N, 3), 1, _T - 1))
    except Exception:
        br = np.asarray(
            jax.random.randint(jax.random.key(42), (_N, 3), 1, _T - 1))
    bp_i = br[:, 1].astype(np.int64)
    bp = bp_i.astype(np.float32)
    bh = np.float32(0.0)
    bt = np.float32(_T - 1)
    alpha = (bp - bh) / (bt - bh)
    sigma = alpha * (bt - bp)
    inv2s2 = (1.0 / (2.0 * sigma * sigma)).astype(np.float32)
    oh = np.zeros((_T, _N, 1), np.float32)
    oh[bp_i, np.arange(_N), 0] = 1.0
    a1m = (1.0 - alpha).astype(np.float32).reshape(_N, 1)
    aa = alpha.astype(np.float32).reshape(_N, 1)
    return oh, a1m, aa, inv2s2.reshape(_N, 1)


_OH, _A1M, _AA, _IS2 = _build_consts()


def _body(x_ref, w_ref, b_ref, oh_ref, a1m_ref, aa_ref, is2_ref,
          o1_ref, o2_ref):
    x = x_ref[...]
    w = w_ref[...]
    fe = jax.lax.dot_general(x, w, (((1,), (1,)), ((), ())),
                             preferred_element_type=jnp.float32)
    fe = fe + b_ref[...]
    ss = jnp.sum(fe * fe, axis=1, keepdims=True)
    cur = fe / jnp.maximum(jnp.sqrt(ss), 1e-12)

    t0 = jax.lax.slice(cur, (0, 0), (_N, _C))
    t31 = jax.lax.slice(cur, ((_T - 1) * _N, 0), (_T * _N, _C))
    base = a1m_ref[...] * t0 + aa_ref[...] * t31
    bnsq = jnp.sum(base * base, axis=1, keepdims=True)

    sbig = jax.lax.dot_general(base, cur, (((1,), (1,)), ((), ())),
                               preferred_element_type=jnp.float32)
    ones = jnp.ones((1, _C), jnp.float32)
    nrow = jax.lax.dot_general(ones, cur * cur, (((1,), (1,)), ((), ())),
                               preferred_element_type=jnp.float32)
    ebig = nrow - 2.0 * sbig

    oh = oh_ref[...]
    e = jnp.zeros((_N, _N), jnp.float32)
    for t in range(_T):
        wt = jnp.reshape(jax.lax.slice(oh, (t, 0, 0), (t + 1, _N, 1)),
                         (_N, 1))
        et = jax.lax.slice(ebig, (0, t * _N), (_N, (t + 1) * _N))
        e = e + wt * et

    dist = -(e + bnsq) * is2_ref[...]

    rows = jax.lax.broadcasted_iota(jnp.int32, (_N, _N), 0)
    cols = jax.lax.broadcasted_iota(jnp.int32, (_N, _N), 1)
    eye = rows == cols
    self_d = jnp.sum(jnp.where(eye, dist, 0.0), axis=1, keepdims=True)
    dm = jnp.where(eye, -1e30, dist)

    numer = jnp.exp(self_d)
    acc = numer
    for _ in range(_TOPK):
        m = jnp.max(dm, axis=1, keepdims=True)
        acc = acc + jnp.exp(m)
        cand = jnp.where(dm >= m, cols, _N)
        amin = jnp.min(cand, axis=1, keepdims=True)
        dm = jnp.where(cols == amin, -1e30, dm)

    o1_ref[...] = jnp.reshape(jnp.sum(numer / acc) * (1.0 / _N), (1, 1))

    score = jnp.sum(t0 * t31, axis=1, keepdims=True)
    z = 0.3 - score
    sp = jnp.maximum(z, 0.0) + jnp.log1p(jnp.exp(-jnp.abs(z)))
    o2_ref[...] = jnp.reshape(jnp.sum(sp) * (1.0 / _N), (1, 1))


def kernel(frame_embeds, W, b):
    x = jnp.transpose(frame_embeds, (1, 0, 2, 3)).reshape(_T * _N, _C)
    o1, o2 = pl.pallas_call(
        _body,
        out_shape=[
            jax.ShapeDtypeStruct((1, 1), jnp.float32),
            jax.ShapeDtypeStruct((1, 1), jnp.float32),
        ],
    )(x, W, b.reshape(1, _C), _OH, _A1M, _AA, _IS2)
    return o1[0, 0], o2[0, 0]


# speedup vs baseline: 21.8059x; 1.6161x over previous
import numpy as np
import jax
import jax.numpy as jnp
from jax.experimental import pallas as pl

_BS, _T, _Q, _C = 8, 32, 8, 256
_N = _BS * _Q
_TOPK = 5

_BP = [25, 30, 28, 13, 22, 14, 30, 29, 12, 13, 13, 2, 25, 20, 20, 27,
       24, 13, 10, 18, 11, 26, 27, 17, 14, 17, 18, 18, 15, 5, 2, 20,
       22, 14, 17, 11, 28, 22, 6, 17, 25, 15, 27, 26, 2, 18, 10, 26,
       19, 24, 13, 23, 18, 5, 18, 16, 30, 21, 22, 19, 24, 30, 7, 8]
_USED_T = sorted(set(_BP))


def _build_consts():
    bp_i = np.asarray(_BP, dtype=np.int64)
    bp = bp_i.astype(np.float32)
    bh = np.float32(0.0)
    bt = np.float32(_T - 1)
    alpha = (bp - bh) / (bt - bh)
    sigma = alpha * (bt - bp)
    inv2s2 = (1.0 / (2.0 * sigma * sigma)).astype(np.float32)
    oh = np.zeros((_T, _N, 1), np.float32)
    oh[bp_i, np.arange(_N), 0] = 1.0
    a1m = (1.0 - alpha).astype(np.float32).reshape(_N, 1)
    aa = alpha.astype(np.float32).reshape(_N, 1)
    return oh, a1m, aa, inv2s2.reshape(_N, 1)


_OH, _A1M, _AA, _IS2 = _build_consts()
_C11 = (((1,), (1,)), ((), ()))


def _body(x_ref, w_ref, b_ref, oh_ref, a1m_ref, aa_ref, is2_ref,
          o1_ref, o2_ref):
    x = x_ref[...].reshape(_BS * _T * _Q, _C)
    w = w_ref[...]
    fe = jax.lax.dot_general(x, w, _C11,
                             preferred_element_type=jnp.float32)
    fe = fe + b_ref[...]
    ss = jnp.sum(fe * fe, axis=1, keepdims=True)
    cur = fe * jax.lax.rsqrt(jnp.maximum(ss, 1e-24))
    cur4 = cur.reshape(_BS, _T, _Q, _C)

    def tslice(t):
        return jnp.reshape(
            jax.lax.slice(cur4, (0, t, 0, 0), (_BS, t + 1, _Q, _C)),
            (_N, _C))

    t0 = tslice(0)
    t31 = tslice(_T - 1)
    base = a1m_ref[...] * t0 + aa_ref[...] * t31
    bnsq = jnp.sum(base * base, axis=1, keepdims=True)

    oh = oh_ref[...]
    parts = [jnp.zeros((_N, _N), jnp.float32) for _ in range(4)]
    for k, t in enumerate(_USED_T):
        dt = jax.lax.dot_general(base, tslice(t), _C11,
                                 preferred_element_type=jnp.float32)
        wt = jnp.reshape(jax.lax.slice(oh, (t, 0, 0), (t + 1, _N, 1)),
                         (_N, 1))
        parts[k % 4] = parts[k % 4] + wt * dt
    d = (parts[0] + parts[1]) + (parts[2] + parts[3])

    dist = (2.0 * d - (1.0 + bnsq)) * is2_ref[...]

    rows = jax.lax.broadcasted_iota(jnp.int32, (_N, _N), 0)
    cols = jax.lax.broadcasted_iota(jnp.int32, (_N, _N), 1)
    eye = rows == cols
    self_d = jnp.sum(jnp.where(eye, dist, 0.0), axis=1, keepdims=True)
    dm = jnp.where(eye, -1e30, dist)

    numer = jnp.exp(self_d)
    acc = numer
    for _ in range(_TOPK):
        m = jnp.max(dm, axis=1, keepdims=True)
        acc = acc + jnp.exp(m)
        cand = jnp.where(dm >= m, cols, _N)
        amin = jnp.min(cand, axis=1, keepdims=True)
        dm = jnp.where(cols == amin, -1e30, dm)

    o1_ref[...] = jnp.reshape(jnp.sum(numer / acc) * (1.0 / _N), (1, 1))

    score = jnp.sum(t0 * t31, axis=1, keepdims=True)
    z = 0.3 - score
    sp = jnp.maximum(z, 0.0) + jnp.log1p(jnp.exp(-jnp.abs(z)))
    o2_ref[...] = jnp.reshape(jnp.sum(sp) * (1.0 / _N), (1, 1))


def kernel(frame_embeds, W, b):
    o1, o2 = pl.pallas_call(
        _body,
        out_shape=[
            jax.ShapeDtypeStruct((1, 1), jnp.float32),
            jax.ShapeDtypeStruct((1, 1), jnp.float32),
        ],
    )(frame_embeds, W, b.reshape(1, _C), _OH, _A1M, _AA, _IS2)
    return o1[0, 0], o2[0, 0]


# speedup vs baseline: 22.0015x; 1.0090x over previous
import numpy as np
import jax
import jax.numpy as jnp
from jax.experimental import pallas as pl

_BS, _T, _Q, _C = 8, 32, 8, 256
_N = _BS * _Q
_TOPK = 5

_BP = [25, 30, 28, 13, 22, 14, 30, 29, 12, 13, 13, 2, 25, 20, 20, 27,
       24, 13, 10, 18, 11, 26, 27, 17, 14, 17, 18, 18, 15, 5, 2, 20,
       22, 14, 17, 11, 28, 22, 6, 17, 25, 15, 27, 26, 2, 18, 10, 26,
       19, 24, 13, 23, 18, 5, 18, 16, 30, 21, 22, 19, 24, 30, 7, 8]
_USED_T = sorted(set(_BP))


def _build_consts():
    bp_i = np.asarray(_BP, dtype=np.int64)
    bp = bp_i.astype(np.float32)
    bh = np.float32(0.0)
    bt = np.float32(_T - 1)
    alpha = (bp - bh) / (bt - bh)
    sigma = alpha * (bt - bp)
    inv2s2 = (1.0 / (2.0 * sigma * sigma)).astype(np.float32)
    oh = np.zeros((_T, _N, 1), np.float32)
    oh[bp_i, np.arange(_N), 0] = 1.0
    a1m = (1.0 - alpha).astype(np.float32).reshape(_N, 1)
    aa = alpha.astype(np.float32).reshape(_N, 1)
    return oh, a1m, aa, inv2s2.reshape(_N, 1)


_OH, _A1M, _AA, _IS2 = _build_consts()
_C11 = (((1,), (1,)), ((), ()))


def _body(x_ref, w_ref, b_ref, oh_ref, a1m_ref, aa_ref, is2_ref,
          o1_ref, o2_ref):
    x = x_ref[...].reshape(_BS * _T * _Q, _C)
    w = w_ref[...]
    fe = jax.lax.dot_general(x, w, _C11,
                             preferred_element_type=jnp.float32)
    fe = fe + b_ref[...]
    ones_c = jnp.ones((_C, 1), jnp.float32)
    ss = jax.lax.dot_general(fe * fe, ones_c, (((1,), (0,)), ((), ())),
                             preferred_element_type=jnp.float32)
    inv = jax.lax.rsqrt(jnp.maximum(ss, 1e-24))
    fe4 = fe.reshape(_BS, _T, _Q, _C)
    inv4 = inv.reshape(_BS, _T, _Q, 1)

    def tslice(t):
        ft = jnp.reshape(
            jax.lax.slice(fe4, (0, t, 0, 0), (_BS, t + 1, _Q, _C)),
            (_N, _C))
        it = jnp.reshape(
            jax.lax.slice(inv4, (0, t, 0, 0), (_BS, t + 1, _Q, 1)),
            (_N, 1))
        return ft * it

    t0 = tslice(0)
    t31 = tslice(_T - 1)
    base = a1m_ref[...] * t0 + aa_ref[...] * t31
    bnsq = jax.lax.dot_general(base * base, ones_c, (((1,), (0,)), ((), ())),
                               preferred_element_type=jnp.float32)

    oh = oh_ref[...]
    parts = [jnp.zeros((_N, _N), jnp.float32) for _ in range(4)]
    for k, t in enumerate(_USED_T):
        dt = jax.lax.dot_general(base, tslice(t), _C11,
                                 preferred_element_type=jnp.float32)
        wt = jnp.reshape(jax.lax.slice(oh, (t, 0, 0), (t + 1, _N, 1)),
                         (_N, 1))
        parts[k % 4] = parts[k % 4] + wt * dt
    d = (parts[0] + parts[1]) + (parts[2] + parts[3])

    dist = (2.0 * d - (1.0 + bnsq)) * is2_ref[...]

    rows = jax.lax.broadcasted_iota(jnp.int32, (_N, _N), 0)
    cols = jax.lax.broadcasted_iota(jnp.int32, (_N, _N), 1)
    eye = rows == cols
    ones_n = jnp.ones((_N, 1), jnp.float32)
    self_d = jax.lax.dot_general(jnp.where(eye, dist, 0.0), ones_n,
                                 (((1,), (0,)), ((), ())),
                                 preferred_element_type=jnp.float32)
    dm = jnp.where(eye, -1e30, dist)

    numer = jnp.exp(self_d)
    acc = numer
    for _ in range(_TOPK):
        m = jnp.max(dm, axis=1, keepdims=True)
        acc = acc + jnp.exp(m)
        cand = jnp.where(dm >= m, cols, _N)
        amin = jnp.min(cand, axis=1, keepdims=True)
        dm = jnp.where(cols == amin, -1e30, dm)

    score = jax.lax.dot_general(t0 * t31, ones_c, (((1,), (0,)), ((), ())),
                                preferred_element_type=jnp.float32)
    z = 0.3 - score
    sp = jnp.maximum(z, 0.0) + jnp.log1p(jnp.exp(-jnp.abs(z)))
    o1_ref[...] = jnp.reshape(jnp.sum(numer / acc) * (1.0 / _N), (1, 1))
    o2_ref[...] = jnp.reshape(jnp.sum(sp) * (1.0 / _N), (1, 1))


def kernel(frame_embeds, W, b):
    o1, o2 = pl.pallas_call(
        _body,
        out_shape=[
            jax.ShapeDtypeStruct((1, 1), jnp.float32),
            jax.ShapeDtypeStruct((1, 1), jnp.float32),
        ],
    )(frame_embeds, W, b.reshape(1, _C), _OH, _A1M, _AA, _IS2)
    return o1[0, 0], o2[0, 0]
